# baseline (speedup 1.0000x reference)
import functools

import jax
import jax.numpy as jnp
from jax import lax
from jax.experimental import pallas as pl
from jax.experimental.pallas import tpu as pltpu
from jax.experimental.pallas import tpu_sc as plsc

NU = 50000
NI = 50000
N = NU + NI
NP = 100096
H = 16
NE = 1600000
B = 16384

ROWS = 12544
PAD_E = ROWS * 128
TPR = ROWS // 16
SB = 16
NSB = TPR // SB
NPT = NP // 16

_MESH = plsc.VectorSubcoreMesh(core_axis_name="c", subcore_axis_name="s")


def _prop_body(src2, dst_rs, vals_rs, e0, zeros, e1, e2, e3,
               src_st, dst_st, vals_st, rv0, rv1, rv2, rv3, dring, acc,
               gs0, gs1, gs2, gs3, ss0, ss1, ss2, ss3, stsem):
    cid = lax.axis_index("c")
    sid = lax.axis_index("s")
    tables = [e0, e1, e2, e3]
    rvs = [rv0, rv1, rv2, rv3]
    gsems = [gs0, gs1, gs2, gs3]
    ssems = [ss0, ss1, ss2, ss3]

    def stage(g, ring_off):
        r0 = sid * TPR + g * SB
        pltpu.async_copy(src2.at[pl.ds(cid * ROWS + r0, SB)],
                         src_st.at[pl.ds(ring_off, SB)], stsem)
        pltpu.async_copy(dst_rs.at[pl.ds(r0, SB)],
                         dst_st.at[pl.ds(ring_off, SB)], stsem)
        pltpu.async_copy(vals_rs.at[pl.ds(r0, SB)],
                         vals_st.at[pl.ds(ring_off, SB)], stsem)

    def wait_stage():
        for _ in range(3):
            pltpu.make_async_copy(src2.at[pl.ds(0, SB)],
                                  src_st.at[pl.ds(0, SB)], stsem).wait()

    def drain_scatter(b):
        pass

    for l in range(3):
        tab = tables[l]
        out = tables[l + 1]

        pltpu.sync_copy(zeros, acc.at[pl.ds(sid * NPT, NPT)])
        plsc.subcore_barrier()

        stage(0, 0)
        wait_stage()

        def block(g, _, tab=tab):
            o = (g % 2) * SB
            o2 = ((g + 1) % 2) * SB

            @pl.when(g > 0)
            def _():
                wait_stage()

            def fire_gather(r, b):
                return pltpu.async_copy(tab.at[src_st.at[o + r]],
                                        rvs[b], gsems[b])

            def drain_then_fire(r, b):
                if r < 4:
                    @pl.when(g > 0)
                    def _():
                        drain_scatter(b)
                else:
                    drain_scatter(b)
                return fire_gather(r, b)

            gd = [None] * SB
            gd[0] = drain_then_fire(0, 0)
            gd[1] = drain_then_fire(1, 1)

            @pl.when(g + 1 < NSB)
            def _():
                stage(g + 1, o2)

            for r in range(SB):
                b = r % 4
                if r + 2 < SB:
                    gd[r + 2] = drain_then_fire(r + 2, (r + 2) % 4)
                gd[r].wait()


            return 0

        lax.fori_loop(0, NSB, block, 0)
        for b in range(4):
            drain_scatter(b)
        plsc.subcore_barrier()


_prop = pl.kernel(
    _prop_body,
    out_type=[jax.ShapeDtypeStruct((2 * NP, 32), jnp.float32)] * 3,
    mesh=_MESH,
    compiler_params=pltpu.CompilerParams(use_tc_tiling_on_sc=False, needs_layout_passes=False),
    scratch_types=[
        pltpu.VMEM((2 * SB, 128), jnp.int32),
        pltpu.VMEM((2 * SB, 128), jnp.int32),
        pltpu.VMEM((2 * SB, 128), jnp.float32),
        pltpu.VMEM((128, 32), jnp.float32),
        pltpu.VMEM((128, 32), jnp.float32),
        pltpu.VMEM((128, 32), jnp.float32),
        pltpu.VMEM((128, 32), jnp.float32),
        pltpu.VMEM((4, 128), jnp.int32),
        pltpu.VMEM_SHARED((NP, H), jnp.float32),
        pltpu.SemaphoreType.DMA,
        pltpu.SemaphoreType.DMA,
        pltpu.SemaphoreType.DMA,
        pltpu.SemaphoreType.DMA,
        pltpu.SemaphoreType.DMA,
        pltpu.SemaphoreType.DMA,
        pltpu.SemaphoreType.DMA,
        pltpu.SemaphoreType.DMA,
        pltpu.SemaphoreType.DMA,
    ],
)


def _batch_body(u_lo, u_hi, i_lo, i_hi, items, xij, wu, wi,
                e0, e1, e2, e3, uxij_t, ix1_t, ix0_t,
                gamma,
                ul_st, uh_st, il_st, ih_st, it_st, xij_st,
                bufs, bux, bx1, bx0, wu_v, wi_v, gamma_v, sem):
    cid = lax.axis_index("c")
    sid = lax.axis_index("s")
    wid = sid * 2 + cid
    base = wid * (B // 32)

    pltpu.sync_copy(wu, wu_v)
    pltpu.sync_copy(wi, wi_v)
    w_u = wu_v[...]
    w_i = wi_v[...]

    def chunk(c, _):
        b0 = base + c * 128
        pltpu.sync_copy(u_lo.at[pl.ds(b0, 128)], ul_st)
        pltpu.sync_copy(u_hi.at[pl.ds(b0, 128)], uh_st)
        pltpu.sync_copy(i_lo.at[pl.ds(b0, 128)], il_st)
        pltpu.sync_copy(i_hi.at[pl.ds(b0, 128)], ih_st)
        pltpu.sync_copy(items.at[pl.ds(b0, 128)], it_st)
        pltpu.sync_copy(xij.at[pl.ds(b0, 128)], xij_st)

        copies = []
        for t, tab in enumerate((e0, e1, e2, e3)):
            copies.append(pltpu.async_copy(tab.at[ul_st], bufs[4 * t + 0], sem))
            copies.append(pltpu.async_copy(tab.at[uh_st], bufs[4 * t + 1], sem))
            copies.append(pltpu.async_copy(tab.at[il_st], bufs[4 * t + 2], sem))
            copies.append(pltpu.async_copy(tab.at[ih_st], bufs[4 * t + 3], sem))
        copies.append(pltpu.async_copy(uxij_t.at[ul_st], bux, sem))
        copies.append(pltpu.async_copy(ix1_t.at[it_st], bx1, sem))
        copies.append(pltpu.async_copy(ix0_t.at[it_st], bx0, sem))
        for cp in copies:
            cp.wait()

        def elem(e, _):
            ue_lo = (bufs[0][e] + bufs[4][e] + bufs[8][e] + bufs[12][e]) * w_u
            ue_hi = (bufs[1][e] + bufs[5][e] + bufs[9][e] + bufs[13][e]) * w_u
            ie_lo = (bufs[2][e] + bufs[6][e] + bufs[10][e] + bufs[14][e]) * w_i
            ie_hi = (bufs[3][e] + bufs[7][e] + bufs[11][e] + bufs[15][e]) * w_i
            uxv = bux[e]
            xf = plsc.load_gather(xij_st, [jnp.full((16,), e, jnp.int32)])
            ixv = bx0[e] + (bx1[e] - bx0[e]) * xf

            m = jnp.max(jnp.maximum(jnp.maximum(ue_lo, ue_hi), uxv))
            el_ = jnp.exp(ue_lo - m)
            eh_ = jnp.exp(ue_hi - m)
            ex_ = jnp.exp(uxv - m)
            s = jnp.sum(el_ + eh_ + ex_)

            sl_ = 1.0 / (1.0 + jnp.exp(-ie_lo))
            sh_ = 1.0 / (1.0 + jnp.exp(-ie_hi))
            sx_ = 1.0 / (1.0 + jnp.exp(-ixv))

            n = jnp.sum(el_ * sl_ + eh_ * sh_ + ex_ * sx_)
            g_vec = jnp.full((16,), n, jnp.float32) / jnp.full((16,), s,
                                                              jnp.float32)
            plsc.store_scatter(gamma_v, [jnp.full((16,), e, jnp.int32)], g_vec)
            return 0

        lax.fori_loop(0, 128, elem, 0)
        pltpu.sync_copy(gamma_v, gamma.at[pl.ds(b0, 128)])
        return 0

    lax.fori_loop(0, B // 32 // 128, chunk, 0)


_batch = pl.kernel(
    _batch_body,
    out_type=jax.ShapeDtypeStruct((B,), jnp.float32),
    mesh=_MESH,
    compiler_params=pltpu.CompilerParams(use_tc_tiling_on_sc=False, needs_layout_passes=False),
    scratch_types=[
        pltpu.VMEM((128,), jnp.int32),
        pltpu.VMEM((128,), jnp.int32),
        pltpu.VMEM((128,), jnp.int32),
        pltpu.VMEM((128,), jnp.int32),
        pltpu.VMEM((128,), jnp.int32),
        pltpu.VMEM((128,), jnp.float32),
        [pltpu.VMEM((128, H), jnp.float32)] * 16,
        pltpu.VMEM((128, H), jnp.float32),
        pltpu.VMEM((128, H), jnp.float32),
        pltpu.VMEM((128, H), jnp.float32),
        pltpu.VMEM((16,), jnp.float32),
        pltpu.VMEM((16,), jnp.float32),
        pltpu.VMEM((128,), jnp.float32),
        pltpu.SemaphoreType.DMA,
    ],
)


def kernel(users, items, xij, edge_index, edge_vals, user_emb, item_emb,
           user_xij, item_xij1, item_xij0, w_user, w_item):
    f32, i32 = jnp.float32, jnp.int32
    users = users.astype(i32)
    items = items.astype(i32)
    xij = xij.astype(i32)
    src = edge_index[0].astype(i32)
    dst = edge_index[1].astype(i32)
    vals = edge_vals.astype(f32)
    user_emb = user_emb.astype(f32)
    item_emb = item_emb.astype(f32)
    user_xij = user_xij.astype(f32)
    item_xij1 = item_xij1.astype(f32)
    item_xij0 = item_xij0.astype(f32)

    all_emb = jnp.concatenate(
        [user_emb, item_emb, jnp.zeros((NP - N, 32), f32)], axis=0)
    e0 = jnp.concatenate([all_emb[:, :H], all_emb[:, H:]], axis=0)

    pad = PAD_E - NE
    srcp = jnp.concatenate([src, jnp.zeros((pad,), i32)])
    dstp = jnp.concatenate([dst, jnp.zeros((pad,), i32)])
    valsp = jnp.concatenate([vals, jnp.zeros((pad,), f32)])
    src2 = jnp.concatenate([srcp, srcp + NP]).reshape(2 * ROWS, 128)
    dst_rs = dstp.reshape(ROWS, 128)
    vals_rs = valsp.reshape(ROWS, 128)
    zeros = jnp.zeros((NPT, H), f32)

    e1, e2, e3 = _prop(src2, dst_rs, vals_rs, jnp.concatenate([e0, e0], axis=1), zeros)

    u_lo = users
    u_hi = users + NP
    i_lo = items + NU
    i_hi = items + NP + NU
    wu = jnp.full((16,), w_user[0].astype(f32) * 0.25, f32)
    wi = jnp.full((16,), w_item[0].astype(f32) * 0.25, f32)

    gamma = _batch(u_lo, u_hi, i_lo, i_hi, items, xij.astype(f32), wu, wi,
                   e0, e1[:, :16], e2[:, :16], e3[:, :16], user_xij, item_xij1, item_xij0)
    return gamma


# speedup vs baseline: 1.4153x; 1.4153x over previous
import functools

import jax
import jax.numpy as jnp
from jax import lax
from jax.experimental import pallas as pl
from jax.experimental.pallas import tpu as pltpu
from jax.experimental.pallas import tpu_sc as plsc

NU = 50000
NI = 50000
N = NU + NI
NP = 100096
H = 16
NE = 1600000
B = 16384

ROWS = 12544
PAD_E = ROWS * 128
TPR = ROWS // 16
SB = 16
NSB = TPR // SB
NPT = NP // 16

_MESH = plsc.VectorSubcoreMesh(core_axis_name="c", subcore_axis_name="s")


def _prop_body(src2, dst_rs, vals_rs, e0, zeros, e1, e2, e3,
               src_st, dst_st, vals_st, rv0, rv1, rv2, rv3, acc,
               gs0, gs1, gs2, gs3, ss0, ss1, ss2, ss3, stsem):
    cid = lax.axis_index("c")
    sid = lax.axis_index("s")
    tables = [e0, e1, e2, e3]
    rvs = [rv0, rv1, rv2, rv3]
    gsems = [gs0, gs1, gs2, gs3]
    ssems = [ss0, ss1, ss2, ss3]

    def stage(g, ring_off):
        r0 = sid * TPR + g * SB
        pltpu.async_copy(src2.at[pl.ds(cid * ROWS + r0, SB)],
                         src_st.at[pl.ds(ring_off, SB)], stsem)
        pltpu.async_copy(dst_rs.at[pl.ds(r0, SB)],
                         dst_st.at[pl.ds(ring_off, SB)], stsem)
        pltpu.async_copy(vals_rs.at[pl.ds(r0, SB)],
                         vals_st.at[pl.ds(ring_off, SB)], stsem)

    def wait_stage():
        for _ in range(3):
            pltpu.make_async_copy(src2.at[pl.ds(0, SB)],
                                  src_st.at[pl.ds(0, SB)], stsem).wait()

    def drain_scatter(b):
        pltpu.make_async_copy(e0.at[pl.ds(0, 128)], rvs[b], ssems[b]).wait()

    for l in range(3):
        tab = tables[l]
        out = tables[l + 1]

        pltpu.sync_copy(zeros, acc.at[pl.ds(sid * NPT, NPT)])
        plsc.subcore_barrier()

        stage(0, 0)
        wait_stage()

        def block(g, _, tab=tab):
            o = (g % 2) * SB
            o2 = ((g + 1) % 2) * SB

            @pl.when(g > 0)
            def _():
                wait_stage()
                for b in range(4):
                    drain_scatter(b)

            def fire_gather(r, b):
                return pltpu.async_copy(tab.at[src_st.at[o + r]],
                                        rvs[b], gsems[b])

            gd = [None] * SB
            gd[0] = fire_gather(0, 0)
            gd[1] = fire_gather(1, 1)

            @pl.when(g + 1 < NSB)
            def _():
                stage(g + 1, o2)

            for r in range(SB):
                b = r % 4
                if r + 2 < SB:
                    nb = (r + 2) % 4
                    if r >= 2:
                        drain_scatter(nb)
                    gd[r + 2] = fire_gather(r + 2, nb)
                gd[r].wait()

                @plsc.parallel_loop(0, 8, unroll=2)
                def _(k, b=b):
                    vv = vals_st[o + r, pl.ds(16 * k, 16)]
                    e0_ = 16 * k
                    for t in range(16):
                        rvs[b][e0_ + t] = rvs[b][e0_ + t] * vv[t]

                pltpu.async_copy(rvs[b], acc.at[dst_st.at[o + r]],
                                 ssems[b], add=True)
            return 0

        lax.fori_loop(0, NSB, block, 0)
        for b in range(4):
            drain_scatter(b)
        plsc.subcore_barrier()
        pltpu.sync_copy(acc.at[pl.ds(sid * NPT, NPT)],
                        out.at[pl.ds(cid * NP + sid * NPT, NPT)])
        plsc.subcore_barrier()


_prop = pl.kernel(
    _prop_body,
    out_type=[jax.ShapeDtypeStruct((2 * NP, H), jnp.float32)] * 3,
    mesh=_MESH,
    compiler_params=pltpu.CompilerParams(use_tc_tiling_on_sc=False, needs_layout_passes=False),
    scratch_types=[
        pltpu.VMEM((2 * SB, 128), jnp.int32),
        pltpu.VMEM((2 * SB, 128), jnp.int32),
        pltpu.VMEM((2 * SB, 128), jnp.float32),
        pltpu.VMEM((128, H), jnp.float32),
        pltpu.VMEM((128, H), jnp.float32),
        pltpu.VMEM((128, H), jnp.float32),
        pltpu.VMEM((128, H), jnp.float32),
        pltpu.VMEM_SHARED((NP, H), jnp.float32),
        pltpu.SemaphoreType.DMA,
        pltpu.SemaphoreType.DMA,
        pltpu.SemaphoreType.DMA,
        pltpu.SemaphoreType.DMA,
        pltpu.SemaphoreType.DMA,
        pltpu.SemaphoreType.DMA,
        pltpu.SemaphoreType.DMA,
        pltpu.SemaphoreType.DMA,
        pltpu.SemaphoreType.DMA,
    ],
)


def _batch_body(u_lo, u_hi, i_lo, i_hi, items, xij, wu, wi,
                e0, e1, e2, e3, uxij_t, ix1_t, ix0_t,
                gamma,
                ul_st, uh_st, il_st, ih_st, it_st, xij_st,
                bufs, bux, bx1, bx0, wu_v, wi_v, gamma_v, sem):
    cid = lax.axis_index("c")
    sid = lax.axis_index("s")
    wid = sid * 2 + cid
    base = wid * (B // 32)

    pltpu.sync_copy(wu, wu_v)
    pltpu.sync_copy(wi, wi_v)
    w_u = wu_v[...]
    w_i = wi_v[...]

    def chunk(c, _):
        b0 = base + c * 128
        pltpu.sync_copy(u_lo.at[pl.ds(b0, 128)], ul_st)
        pltpu.sync_copy(u_hi.at[pl.ds(b0, 128)], uh_st)
        pltpu.sync_copy(i_lo.at[pl.ds(b0, 128)], il_st)
        pltpu.sync_copy(i_hi.at[pl.ds(b0, 128)], ih_st)
        pltpu.sync_copy(items.at[pl.ds(b0, 128)], it_st)
        pltpu.sync_copy(xij.at[pl.ds(b0, 128)], xij_st)

        copies = []
        for t, tab in enumerate((e0, e1, e2, e3)):
            copies.append(pltpu.async_copy(tab.at[ul_st], bufs[4 * t + 0], sem))
            copies.append(pltpu.async_copy(tab.at[uh_st], bufs[4 * t + 1], sem))
            copies.append(pltpu.async_copy(tab.at[il_st], bufs[4 * t + 2], sem))
            copies.append(pltpu.async_copy(tab.at[ih_st], bufs[4 * t + 3], sem))
        copies.append(pltpu.async_copy(uxij_t.at[ul_st], bux, sem))
        copies.append(pltpu.async_copy(ix1_t.at[it_st], bx1, sem))
        copies.append(pltpu.async_copy(ix0_t.at[it_st], bx0, sem))
        for cp in copies:
            cp.wait()

        def elem(e, _):
            ue_lo = (bufs[0][e] + bufs[4][e] + bufs[8][e] + bufs[12][e]) * w_u
            ue_hi = (bufs[1][e] + bufs[5][e] + bufs[9][e] + bufs[13][e]) * w_u
            ie_lo = (bufs[2][e] + bufs[6][e] + bufs[10][e] + bufs[14][e]) * w_i
            ie_hi = (bufs[3][e] + bufs[7][e] + bufs[11][e] + bufs[15][e]) * w_i
            uxv = bux[e]
            xf = plsc.load_gather(xij_st, [jnp.full((16,), e, jnp.int32)])
            ixv = bx0[e] + (bx1[e] - bx0[e]) * xf

            m = jnp.max(jnp.maximum(jnp.maximum(ue_lo, ue_hi), uxv))
            el_ = jnp.exp(ue_lo - m)
            eh_ = jnp.exp(ue_hi - m)
            ex_ = jnp.exp(uxv - m)
            s = jnp.sum(el_ + eh_ + ex_)

            sl_ = 1.0 / (1.0 + jnp.exp(-ie_lo))
            sh_ = 1.0 / (1.0 + jnp.exp(-ie_hi))
            sx_ = 1.0 / (1.0 + jnp.exp(-ixv))

            n = jnp.sum(el_ * sl_ + eh_ * sh_ + ex_ * sx_)
            g_vec = jnp.full((16,), n, jnp.float32) / jnp.full((16,), s,
                                                              jnp.float32)
            plsc.store_scatter(gamma_v, [jnp.full((16,), e, jnp.int32)], g_vec)
            return 0

        lax.fori_loop(0, 128, elem, 0)
        pltpu.sync_copy(gamma_v, gamma.at[pl.ds(b0, 128)])
        return 0

    lax.fori_loop(0, B // 32 // 128, chunk, 0)


_batch = pl.kernel(
    _batch_body,
    out_type=jax.ShapeDtypeStruct((B,), jnp.float32),
    mesh=_MESH,
    compiler_params=pltpu.CompilerParams(use_tc_tiling_on_sc=False, needs_layout_passes=False),
    scratch_types=[
        pltpu.VMEM((128,), jnp.int32),
        pltpu.VMEM((128,), jnp.int32),
        pltpu.VMEM((128,), jnp.int32),
        pltpu.VMEM((128,), jnp.int32),
        pltpu.VMEM((128,), jnp.int32),
        pltpu.VMEM((128,), jnp.float32),
        [pltpu.VMEM((128, H), jnp.float32)] * 16,
        pltpu.VMEM((128, H), jnp.float32),
        pltpu.VMEM((128, H), jnp.float32),
        pltpu.VMEM((128, H), jnp.float32),
        pltpu.VMEM((16,), jnp.float32),
        pltpu.VMEM((16,), jnp.float32),
        pltpu.VMEM((128,), jnp.float32),
        pltpu.SemaphoreType.DMA,
    ],
)


def kernel(users, items, xij, edge_index, edge_vals, user_emb, item_emb,
           user_xij, item_xij1, item_xij0, w_user, w_item):
    f32, i32 = jnp.float32, jnp.int32
    users = users.astype(i32)
    items = items.astype(i32)
    xij = xij.astype(i32)
    src = edge_index[0].astype(i32)
    dst = edge_index[1].astype(i32)
    vals = edge_vals.astype(f32)
    user_emb = user_emb.astype(f32)
    item_emb = item_emb.astype(f32)
    user_xij = user_xij.astype(f32)
    item_xij1 = item_xij1.astype(f32)
    item_xij0 = item_xij0.astype(f32)

    all_emb = jnp.concatenate(
        [user_emb, item_emb, jnp.zeros((NP - N, 32), f32)], axis=0)
    e0 = jnp.concatenate([all_emb[:, :H], all_emb[:, H:]], axis=0)

    pad = PAD_E - NE
    srcp = jnp.concatenate([src, jnp.zeros((pad,), i32)])
    dstp = jnp.concatenate([dst, jnp.zeros((pad,), i32)])
    valsp = jnp.concatenate([vals, jnp.zeros((pad,), f32)])
    src2 = jnp.concatenate([srcp, srcp + NP]).reshape(2 * ROWS, 128)
    dst_rs = dstp.reshape(ROWS, 128)
    vals_rs = valsp.reshape(ROWS, 128)
    zeros = jnp.zeros((NPT, H), f32)

    e1, e2, e3 = _prop(src2, dst_rs, vals_rs, e0, zeros)

    u_lo = users
    u_hi = users + NP
    i_lo = items + NU
    i_hi = items + NP + NU
    wu = jnp.full((16,), w_user[0].astype(f32) * 0.25, f32)
    wi = jnp.full((16,), w_item[0].astype(f32) * 0.25, f32)

    gamma = _batch(u_lo, u_hi, i_lo, i_hi, items, xij.astype(f32), wu, wi,
                   e0, e1, e2, e3, user_xij, item_xij1, item_xij0)
    return gamma


# speedup vs baseline: 1.6069x; 1.1354x over previous
import functools

import jax
import jax.numpy as jnp
from jax import lax
from jax.experimental import pallas as pl
from jax.experimental.pallas import tpu as pltpu
from jax.experimental.pallas import tpu_sc as plsc

NU = 50000
NI = 50000
N = NU + NI
NP = 100096
H = 16
NE = 1600000
B = 16384

ROWS = 12544
PAD_E = ROWS * 128
TPR = ROWS // 16
SB = 16
NSB = TPR // SB
NPT = NP // 16

_MESH = plsc.VectorSubcoreMesh(core_axis_name="c", subcore_axis_name="s")


def _prop_body(src2, dst_rs, vals_rs, e0, zeros, e1, e2, e3,
               src_st, dst_st, vals_st,
               rv0, rv1, rv2, rv3, rv4, rv5, rv6, rv7, acc,
               gs0, gs1, gs2, gs3, gs4, gs5, gs6, gs7,
               ss0, ss1, ss2, ss3, ss4, ss5, ss6, ss7, stsem):
    cid = lax.axis_index("c")
    sid = lax.axis_index("s")
    tables = [e0, e1, e2, e3]
    rvs = [rv0, rv1, rv2, rv3, rv4, rv5, rv6, rv7]
    gsems = [gs0, gs1, gs2, gs3, gs4, gs5, gs6, gs7]
    ssems = [ss0, ss1, ss2, ss3, ss4, ss5, ss6, ss7]

    def stage(g, ring_off):
        r0 = sid * TPR + g * SB
        pltpu.async_copy(src2.at[pl.ds(cid * ROWS + r0, SB)],
                         src_st.at[pl.ds(ring_off, SB)], stsem)
        pltpu.async_copy(dst_rs.at[pl.ds(r0, SB)],
                         dst_st.at[pl.ds(ring_off, SB)], stsem)
        pltpu.async_copy(vals_rs.at[pl.ds(r0, SB)],
                         vals_st.at[pl.ds(ring_off, SB)], stsem)

    def wait_stage():
        for _ in range(3):
            pltpu.make_async_copy(src2.at[pl.ds(0, SB)],
                                  src_st.at[pl.ds(0, SB)], stsem).wait()

    def drain_scatter(b):
        pltpu.make_async_copy(e0.at[pl.ds(0, 128)], rvs[b], ssems[b]).wait()

    for l in range(3):
        tab = tables[l]
        out = tables[l + 1]

        pltpu.sync_copy(zeros, acc.at[pl.ds(sid * NPT, NPT)])
        plsc.subcore_barrier()

        stage(0, 0)
        wait_stage()

        def block(g, _, tab=tab):
            o = (g % 2) * SB
            o2 = ((g + 1) % 2) * SB

            @pl.when(g > 0)
            def _():
                wait_stage()

            def fire_gather(r, b):
                return pltpu.async_copy(tab.at[src_st.at[o + r]],
                                        rvs[b], gsems[b])

            def drain_then_fire(r, b):
                if r < 8:
                    @pl.when(g > 0)
                    def _():
                        drain_scatter(b)
                else:
                    drain_scatter(b)
                return fire_gather(r, b)

            gd = [None] * SB
            gd[0] = drain_then_fire(0, 0)
            gd[1] = drain_then_fire(1, 1)
            gd[2] = drain_then_fire(2, 2)

            @pl.when(g + 1 < NSB)
            def _():
                stage(g + 1, o2)

            for r in range(SB):
                b = r % 8
                if r + 3 < SB:
                    gd[r + 3] = drain_then_fire(r + 3, (r + 3) % 8)
                gd[r].wait()

                @plsc.parallel_loop(0, 8, unroll=2)
                def _(k, b=b):
                    vv = vals_st[o + r, pl.ds(16 * k, 16)]
                    e0_ = 16 * k
                    for t in range(16):
                        rvs[b][e0_ + t] = rvs[b][e0_ + t] * vv[t]

                pltpu.async_copy(rvs[b], acc.at[dst_st.at[o + r]],
                                 ssems[b], add=True)
            return 0

        lax.fori_loop(0, NSB, block, 0)
        for b in range(8):
            drain_scatter(b)
        plsc.subcore_barrier()
        pltpu.sync_copy(acc.at[pl.ds(sid * NPT, NPT)],
                        out.at[pl.ds(cid * NP + sid * NPT, NPT)])
        plsc.subcore_barrier()


_prop = pl.kernel(
    _prop_body,
    out_type=[jax.ShapeDtypeStruct((2 * NP, H), jnp.float32)] * 3,
    mesh=_MESH,
    compiler_params=pltpu.CompilerParams(use_tc_tiling_on_sc=False, needs_layout_passes=False),
    scratch_types=[
        pltpu.VMEM((2 * SB, 128), jnp.int32),
        pltpu.VMEM((2 * SB, 128), jnp.int32),
        pltpu.VMEM((2 * SB, 128), jnp.float32),
        pltpu.VMEM((128, H), jnp.float32),
        pltpu.VMEM((128, H), jnp.float32),
        pltpu.VMEM((128, H), jnp.float32),
        pltpu.VMEM((128, H), jnp.float32),
        pltpu.VMEM((128, H), jnp.float32),
        pltpu.VMEM((128, H), jnp.float32),
        pltpu.VMEM((128, H), jnp.float32),
        pltpu.VMEM((128, H), jnp.float32),
        pltpu.VMEM_SHARED((NP, H), jnp.float32),
    ] + [pltpu.SemaphoreType.DMA] * 17,
)


def _batch_body(u_lo, u_hi, i_lo, i_hi, items, xij, wu, wi,
                e0, e1, e2, e3, uxij_t, ix1_t, ix0_t,
                gamma,
                ul_st, uh_st, il_st, ih_st, it_st, xij_st,
                bufs, bux, bx1, bx0, wu_v, wi_v, gamma_v, sem):
    cid = lax.axis_index("c")
    sid = lax.axis_index("s")
    wid = sid * 2 + cid
    base = wid * (B // 32)

    pltpu.sync_copy(wu, wu_v)
    pltpu.sync_copy(wi, wi_v)
    w_u = wu_v[...]
    w_i = wi_v[...]

    def chunk(c, _):
        b0 = base + c * 128
        pltpu.sync_copy(u_lo.at[pl.ds(b0, 128)], ul_st)
        pltpu.sync_copy(u_hi.at[pl.ds(b0, 128)], uh_st)
        pltpu.sync_copy(i_lo.at[pl.ds(b0, 128)], il_st)
        pltpu.sync_copy(i_hi.at[pl.ds(b0, 128)], ih_st)
        pltpu.sync_copy(items.at[pl.ds(b0, 128)], it_st)
        pltpu.sync_copy(xij.at[pl.ds(b0, 128)], xij_st)

        copies = []
        for t, tab in enumerate((e0, e1, e2, e3)):
            copies.append(pltpu.async_copy(tab.at[ul_st], bufs[4 * t + 0], sem))
            copies.append(pltpu.async_copy(tab.at[uh_st], bufs[4 * t + 1], sem))
            copies.append(pltpu.async_copy(tab.at[il_st], bufs[4 * t + 2], sem))
            copies.append(pltpu.async_copy(tab.at[ih_st], bufs[4 * t + 3], sem))
        copies.append(pltpu.async_copy(uxij_t.at[ul_st], bux, sem))
        copies.append(pltpu.async_copy(ix1_t.at[it_st], bx1, sem))
        copies.append(pltpu.async_copy(ix0_t.at[it_st], bx0, sem))
        for cp in copies:
            cp.wait()

        def elem(e, _):
            ue_lo = (bufs[0][e] + bufs[4][e] + bufs[8][e] + bufs[12][e]) * w_u
            ue_hi = (bufs[1][e] + bufs[5][e] + bufs[9][e] + bufs[13][e]) * w_u
            ie_lo = (bufs[2][e] + bufs[6][e] + bufs[10][e] + bufs[14][e]) * w_i
            ie_hi = (bufs[3][e] + bufs[7][e] + bufs[11][e] + bufs[15][e]) * w_i
            uxv = bux[e]
            xf = plsc.load_gather(xij_st, [jnp.full((16,), e, jnp.int32)])
            ixv = bx0[e] + (bx1[e] - bx0[e]) * xf

            m = jnp.max(jnp.maximum(jnp.maximum(ue_lo, ue_hi), uxv))
            el_ = jnp.exp(ue_lo - m)
            eh_ = jnp.exp(ue_hi - m)
            ex_ = jnp.exp(uxv - m)
            s = jnp.sum(el_ + eh_ + ex_)

            sl_ = 1.0 / (1.0 + jnp.exp(-ie_lo))
            sh_ = 1.0 / (1.0 + jnp.exp(-ie_hi))
            sx_ = 1.0 / (1.0 + jnp.exp(-ixv))

            n = jnp.sum(el_ * sl_ + eh_ * sh_ + ex_ * sx_)
            g_vec = jnp.full((16,), n, jnp.float32) / jnp.full((16,), s,
                                                              jnp.float32)
            plsc.store_scatter(gamma_v, [jnp.full((16,), e, jnp.int32)], g_vec)
            return 0

        lax.fori_loop(0, 128, elem, 0)
        pltpu.sync_copy(gamma_v, gamma.at[pl.ds(b0, 128)])
        return 0

    lax.fori_loop(0, B // 32 // 128, chunk, 0)


_batch = pl.kernel(
    _batch_body,
    out_type=jax.ShapeDtypeStruct((B,), jnp.float32),
    mesh=_MESH,
    compiler_params=pltpu.CompilerParams(use_tc_tiling_on_sc=False, needs_layout_passes=False),
    scratch_types=[
        pltpu.VMEM((128,), jnp.int32),
        pltpu.VMEM((128,), jnp.int32),
        pltpu.VMEM((128,), jnp.int32),
        pltpu.VMEM((128,), jnp.int32),
        pltpu.VMEM((128,), jnp.int32),
        pltpu.VMEM((128,), jnp.float32),
        [pltpu.VMEM((128, H), jnp.float32)] * 16,
        pltpu.VMEM((128, H), jnp.float32),
        pltpu.VMEM((128, H), jnp.float32),
        pltpu.VMEM((128, H), jnp.float32),
        pltpu.VMEM((16,), jnp.float32),
        pltpu.VMEM((16,), jnp.float32),
        pltpu.VMEM((128,), jnp.float32),
        pltpu.SemaphoreType.DMA,
    ],
)


def kernel(users, items, xij, edge_index, edge_vals, user_emb, item_emb,
           user_xij, item_xij1, item_xij0, w_user, w_item):
    f32, i32 = jnp.float32, jnp.int32
    users = users.astype(i32)
    items = items.astype(i32)
    xij = xij.astype(i32)
    src = edge_index[0].astype(i32)
    dst = edge_index[1].astype(i32)
    vals = edge_vals.astype(f32)
    user_emb = user_emb.astype(f32)
    item_emb = item_emb.astype(f32)
    user_xij = user_xij.astype(f32)
    item_xij1 = item_xij1.astype(f32)
    item_xij0 = item_xij0.astype(f32)

    all_emb = jnp.concatenate(
        [user_emb, item_emb, jnp.zeros((NP - N, 32), f32)], axis=0)
    e0 = jnp.concatenate([all_emb[:, :H], all_emb[:, H:]], axis=0)

    pad = PAD_E - NE
    srcp = jnp.concatenate([src, jnp.zeros((pad,), i32)])
    dstp = jnp.concatenate([dst, jnp.zeros((pad,), i32)])
    valsp = jnp.concatenate([vals, jnp.zeros((pad,), f32)])
    src2 = jnp.concatenate([srcp, srcp + NP]).reshape(2 * ROWS, 128)
    dst_rs = dstp.reshape(ROWS, 128)
    vals_rs = valsp.reshape(ROWS, 128)
    zeros = jnp.zeros((NPT, H), f32)

    e1, e2, e3 = _prop(src2, dst_rs, vals_rs, e0, zeros)

    u_lo = users
    u_hi = users + NP
    i_lo = items + NU
    i_hi = items + NP + NU
    wu = jnp.full((16,), w_user[0].astype(f32) * 0.25, f32)
    wi = jnp.full((16,), w_item[0].astype(f32) * 0.25, f32)

    gamma = _batch(u_lo, u_hi, i_lo, i_hi, items, xij.astype(f32), wu, wi,
                   e0, e1, e2, e3, user_xij, item_xij1, item_xij0)
    return gamma
